# Initial kernel scaffold; baseline (speedup 1.0000x reference)
import functools

import jax
import jax.numpy as jnp
from jax import lax
from jax.experimental import pallas as pl
from jax.experimental.pallas import tpu as pltpu
from jax.experimental.pallas import tpu_sc as plsc

_N = 10000
_E = 320000
_D = 128
_H = 128
_C = 10
_G = 64

_NW = 32
_B = 128
_K = 79
_EPAD = _NW * _K * _B
_NPAD = 10240
_CH = _NPAD // 16


def _sc_body(x_hbm, src_hbm, dst_hbm, z2_hbm, z1_hbm, on_hbm,
             agg_out, deg_out,
             src_v, dst_v, rows_v, ones_v, agg_sh, deg_sh):
    cid = lax.axis_index("c")
    sid = lax.axis_index("s")
    wid = cid * 16 + sid
    base = sid * _CH

    for k in range(_CH // 128):
        pltpu.sync_copy(z2_hbm, agg_sh.at[pl.ds(base + k * 128, 128)])
    pltpu.sync_copy(z1_hbm, deg_sh.at[pl.ds(base, _CH)])
    pltpu.sync_copy(on_hbm, ones_v)
    pltpu.sync_copy(src_hbm.at[wid], src_v)
    pltpu.sync_copy(dst_hbm.at[wid], dst_v)
    plsc.subcore_barrier()

    def body(j, carry):
        pltpu.sync_copy(x_hbm.at[src_v.at[j]], rows_v)
        pltpu.sync_copy(rows_v, agg_sh.at[dst_v.at[j]], add=True)
        pltpu.sync_copy(ones_v, deg_sh.at[dst_v.at[j]], add=True)
        return carry

    lax.fori_loop(0, _K, body, 0)
    plsc.subcore_barrier()

    pltpu.sync_copy(agg_sh.at[pl.ds(base, _CH)],
                    agg_out.at[cid, pl.ds(base, _CH)])
    pltpu.sync_copy(deg_sh.at[pl.ds(base, _CH)],
                    deg_out.at[cid, pl.ds(base, _CH)])


_sc_call = pl.kernel(
    _sc_body,
    out_type=(
        jax.ShapeDtypeStruct((2, _NPAD, _D), jnp.float32),
        jax.ShapeDtypeStruct((2, _NPAD), jnp.float32),
    ),
    mesh=plsc.VectorSubcoreMesh(core_axis_name="c", subcore_axis_name="s"),
    scratch_types=[
        pltpu.VMEM((_K, _B), jnp.int32),
        pltpu.VMEM((_K, _B), jnp.int32),
        pltpu.VMEM((_B, _D), jnp.float32),
        pltpu.VMEM((_B,), jnp.float32),
        pltpu.VMEM_SHARED((_NPAD, _D), jnp.float32),
        pltpu.VMEM_SHARED((_NPAD,), jnp.float32),
    ],
)


def _tc_body(agg_ref, deg_ref, x_ref, b_ref, wc_ref, bc_ref, g_ref, be_ref,
             wl_ref, bl_ref, o_ref):
    agg = agg_ref[0] + agg_ref[1]
    deg = deg_ref[0] + deg_ref[1]
    scale = 1.0 / jnp.maximum(deg, 1.0)
    a = (agg * jnp.reshape(scale, (_NPAD, 1)))[:_N]
    z = jnp.dot(a, wc_ref[...], preferred_element_type=jnp.float32)
    z = z + bc_ref[...]
    mu = jnp.mean(z, axis=0, keepdims=True)
    var = jnp.mean((z - mu) ** 2, axis=0, keepdims=True)
    h = (z - mu) / jnp.sqrt(var + 1e-5) * g_ref[...] + be_ref[...]
    h = h + x_ref[...]
    m = (b_ref[...] == lax.broadcasted_iota(jnp.int32, (_N, _G), 1))
    m = m.astype(jnp.float32)
    sums = lax.dot_general(m, h, (((0,), (0,)), ((), ())),
                           preferred_element_type=jnp.float32)
    cnt = lax.dot_general(m, jnp.ones((_N, 1), jnp.float32),
                          (((0,), (0,)), ((), ())),
                          preferred_element_type=jnp.float32)
    pooled = sums / jnp.maximum(cnt, 1.0)
    o_ref[...] = jnp.dot(pooled, wl_ref[...],
                         preferred_element_type=jnp.float32) + bl_ref[...]


_tc_call = pl.pallas_call(
    _tc_body,
    out_shape=jax.ShapeDtypeStruct((_G, _C), jnp.float32),
)


def kernel(x, edge_index, batch, W_conv, b_conv, gamma, beta, W_lin, b_lin):
    src = edge_index[0]
    dst = edge_index[1]
    pad = _EPAD - _E
    src_p = jnp.concatenate([src, jnp.zeros((pad,), jnp.int32)])
    dst_p = jnp.concatenate([dst, jnp.full((pad,), _N, jnp.int32)])
    src_p = src_p.reshape(_NW, _K, _B)
    dst_p = dst_p.reshape(_NW, _K, _B)
    z2 = jnp.zeros((128, _D), jnp.float32)
    z1 = jnp.zeros((_CH,), jnp.float32)
    on1 = jnp.ones((_B,), jnp.float32)
    agg2, deg2 = _sc_call(x, src_p, dst_p, z2, z1, on1)
    return _tc_call(agg2, deg2, x, batch.reshape(_N, 1), W_conv,
                    b_conv.reshape(1, _H), gamma.reshape(1, _H),
                    beta.reshape(1, _H), W_lin, b_lin.reshape(1, _C))


# speedup vs baseline: 5.8668x; 5.8668x over previous
import functools

import jax
import jax.numpy as jnp
from jax import lax
from jax.experimental import pallas as pl
from jax.experimental.pallas import tpu as pltpu
from jax.experimental.pallas import tpu_sc as plsc

_N = 10000
_E = 320000
_D = 128
_H = 128
_C = 10
_G = 64

_NW = 32
_B = 128
_K = 79
_EPAD = _NW * _K * _B
_NPAD = 10240
_CH = _NPAD // 16


def _sc_body(x_hbm, src_hbm, dst_hbm, z2_hbm, z1_hbm, on_hbm,
             agg_out, deg_out,
             src_v, dst_v, rows_v, ones_v, agg_sh, deg_sh):
    cid = lax.axis_index("c")
    sid = lax.axis_index("s")
    wid = cid * 16 + sid
    base = sid * _CH

    for k in range(_CH // 128):
        pltpu.sync_copy(z2_hbm, agg_sh.at[pl.ds(base + k * 128, 128)])
    pltpu.sync_copy(z1_hbm, deg_sh.at[pl.ds(base, _CH)])
    pltpu.sync_copy(on_hbm, ones_v)
    pltpu.sync_copy(src_hbm.at[wid], src_v)
    pltpu.sync_copy(dst_hbm.at[wid], dst_v)
    plsc.subcore_barrier()

    def body(j, carry):
        pltpu.sync_copy(x_hbm.at[src_v.at[j]], rows_v)
        pltpu.sync_copy(rows_v, agg_sh.at[dst_v.at[j]], add=True)
        pltpu.sync_copy(ones_v, deg_sh.at[dst_v.at[j]], add=True)
        return carry

    lax.fori_loop(0, _K, body, 0)
    plsc.subcore_barrier()

    pltpu.sync_copy(agg_sh.at[pl.ds(base, _CH)],
                    agg_out.at[cid, pl.ds(base, _CH)])
    pltpu.sync_copy(deg_sh.at[pl.ds(base, _CH)],
                    deg_out.at[cid, pl.ds(base, _CH)])


@functools.cache
def _sc_call():
    return pl.kernel(
        _sc_body,
        out_type=(
            jax.ShapeDtypeStruct((2, _NPAD, _D), jnp.float32),
            jax.ShapeDtypeStruct((2, _NPAD), jnp.float32),
        ),
        mesh=plsc.VectorSubcoreMesh(core_axis_name="c", subcore_axis_name="s"),
        scratch_types=[
            pltpu.VMEM((_K, _B), jnp.int32),
            pltpu.VMEM((_K, _B), jnp.int32),
            pltpu.VMEM((_B, _D), jnp.float32),
            pltpu.VMEM((_B,), jnp.float32),
            pltpu.VMEM_SHARED((_NPAD, _D), jnp.float32),
            pltpu.VMEM_SHARED((_NPAD,), jnp.float32),
        ],
    )


def _tc_body(agg_ref, deg_ref, x_ref, b_ref, wc_ref, bc_ref, g_ref, be_ref,
             wl_ref, bl_ref, o_ref):
    agg = agg_ref[0] + agg_ref[1]
    deg = deg_ref[0] + deg_ref[1]
    scale = 1.0 / jnp.maximum(deg, 1.0)
    a = (agg * jnp.reshape(scale, (_NPAD, 1)))[:_N]
    z = jnp.dot(a, wc_ref[...], preferred_element_type=jnp.float32)
    z = z + bc_ref[...]
    mu = jnp.mean(z, axis=0, keepdims=True)
    var = jnp.mean((z - mu) ** 2, axis=0, keepdims=True)
    h = (z - mu) / jnp.sqrt(var + 1e-5) * g_ref[...] + be_ref[...]
    h = h + x_ref[...]
    m = (b_ref[...] == lax.broadcasted_iota(jnp.int32, (_N, _G), 1))
    m = m.astype(jnp.float32)
    sums = lax.dot_general(m, h, (((0,), (0,)), ((), ())),
                           preferred_element_type=jnp.float32)
    cnt = lax.dot_general(m, jnp.ones((_N, 1), jnp.float32),
                          (((0,), (0,)), ((), ())),
                          preferred_element_type=jnp.float32)
    pooled = sums / jnp.maximum(cnt, 1.0)
    o_ref[...] = jnp.dot(pooled, wl_ref[...],
                         preferred_element_type=jnp.float32) + bl_ref[...]


_tc_call = pl.pallas_call(
    _tc_body,
    out_shape=jax.ShapeDtypeStruct((_G, _C), jnp.float32),
)


def kernel(x, edge_index, batch, W_conv, b_conv, gamma, beta, W_lin, b_lin):
    src = edge_index[0]
    dst = edge_index[1]
    pad = _EPAD - _E
    src_p = jnp.concatenate([src, jnp.zeros((pad,), jnp.int32)])
    dst_p = jnp.concatenate([dst, jnp.full((pad,), _N, jnp.int32)])
    src_p = src_p.reshape(_NW, _K, _B)
    dst_p = dst_p.reshape(_NW, _K, _B)
    z2 = jnp.zeros((128, _D), jnp.float32)
    z1 = jnp.zeros((_CH,), jnp.float32)
    on1 = jnp.ones((_B,), jnp.float32)
    agg2, deg2 = _sc_call()(x, src_p, dst_p, z2, z1, on1)
    return _tc_call(agg2, deg2, x, batch.reshape(_N, 1), W_conv,
                    b_conv.reshape(1, _H), gamma.reshape(1, _H),
                    beta.reshape(1, _H), W_lin, b_lin.reshape(1, _C))
